# baseline (speedup 1.0000x reference)
import functools

import jax
import jax.numpy as jnp
from jax import lax
from jax.experimental import pallas as pl
from jax.experimental.pallas import tpu as pltpu
from jax.experimental.pallas import tpu_sc as plsc

EPS = 1e-10
L = 16
NC = 2
NS = 16
NW = NC * NS
C = 256


def _uniform01(x):
    x = x ^ lax.shift_right_logical(x, 16)
    x = x * jnp.int32(-2048144789)
    x = x ^ lax.shift_right_logical(x, 13)
    x = x * jnp.int32(-1028477387)
    x = x ^ lax.shift_right_logical(x, 16)
    bits = lax.shift_right_logical(x, 9) | jnp.int32(0x3F800000)
    return plsc.bitcast(bits, jnp.float32) - jnp.float32(1.0)


def _make_sc_call(n, et):
    assert et == L
    rpt = ((n + NW - 1) // NW + C - 1) // C * C
    nchunk = rpt // C
    groups = C // L
    last_base = (n - rpt) // 128 * 128
    tail_base = last_base + rpt
    ntail = n - tail_base
    assert 0 <= ntail < 128 and ntail % (2 * L) == 0 and groups % 4 == 0
    mesh = plsc.VectorSubcoreMesh(
        core_axis_name="c", subcore_axis_name="s", num_cores=NC,
        num_subcores=NS)

    @functools.partial(
        pl.kernel,
        out_type=(
            jax.ShapeDtypeStruct((NW * L,), jnp.float32),
            jax.ShapeDtypeStruct((L,), jnp.float32),
        ),
        mesh=mesh,
        compiler_params=pltpu.CompilerParams(needs_layout_passes=False),
        scratch_types=[
            pltpu.VMEM((L, C), jnp.float32),
            pltpu.VMEM((L, C), jnp.float32),
            pltpu.VMEM((L, 128), jnp.float32),
            pltpu.VMEM((2 * L,), jnp.float32),
            pltpu.VMEM((2 * L,), jnp.float32),
            pltpu.VMEM((L,), jnp.float32),
            pltpu.VMEM((L,), jnp.float32),
            pltpu.SemaphoreType.DMA,
            pltpu.SemaphoreType.DMA,
        ],
    )
    def sc_call(edge_hbm, ntp_hbm, tail_hbm,
                parts_hbm, pnode_hbm,
                buf0, buf1, first_v, table_v, ntp_v, red_v,
                scal_v, sem0, sem1):
        wid = lax.axis_index("s") * NC + lax.axis_index("c")
        base = jnp.minimum(wid * rpt, last_base)
        bufs = (buf0, buf1)
        sems = (sem0, sem1)

        def issue(chunk, b):
            pltpu.async_copy(
                edge_hbm.at[:, pl.ds(base + chunk * C, C)], bufs[b], sems[b])

        def wait(b):
            pltpu.make_async_copy(
                edge_hbm.at[:, pl.ds(0, C)], bufs[b], sems[b]).wait()

        issue(0, 0)
        issue(1, 1)
        pltpu.sync_copy(edge_hbm.at[:, pl.ds(0, 128)], first_v)

        lanes = lax.iota(jnp.int32, L)
        ones_f = jnp.ones((L,), jnp.float32)
        ones_i = jnp.ones((L,), jnp.int32)
        zeros_i = jnp.zeros((L,), jnp.int32)

        colsum = jnp.zeros((L,), jnp.float32)
        colprod = ones_f
        for j in range(L):
            c = first_v[j, pl.ds(0, L)]
            colsum = colsum + c
            colprod = colprod * c
        rinv = jnp.float32(1.0) / (colsum + EPS)
        r2 = rinv * rinv
        r4 = r2 * r2
        r8 = r4 * r4
        table_v[pl.ds(0, L)] = colprod * (r8 * r8)
        table_v[pl.ds(L, L)] = ones_f

        def group_factor(buf, rowbase, off):
            u = _uniform01(rowbase + off + lanes)
            cols = [buf[j, pl.ds(off, L)] for j in range(L)]
            prefs = [cols[0]]
            for j in range(1, L):
                prefs.append(prefs[-1] + cols[j])
            t = u * prefs[-1]
            bits = [jnp.where(pj < t, ones_i, zeros_i) for pj in prefs]
            while len(bits) > 1:
                bits = [bits[j] + bits[j + 1] for j in range(0, len(bits), 2)]
            return plsc.load_gather(table_v, [bits[0]])

        def quad_body(buf, rowbase, g4, acc):
            f0 = group_factor(buf, rowbase, (4 * g4) * L)
            f1 = group_factor(buf, rowbase, (4 * g4 + 1) * L)
            f2 = group_factor(buf, rowbase, (4 * g4 + 2) * L)
            f3 = group_factor(buf, rowbase, (4 * g4 + 3) * L)
            return acc * ((f0 * f1) * (f2 * f3))

        def compute_chunk(b, chunk, acc):
            wait(b)
            rowbase = base + chunk * C
            acc = lax.fori_loop(
                0, groups // 4,
                lambda g4, a: quad_body(bufs[b], rowbase, g4, a), acc)
            return acc

        def outer(o, acc):
            for b in (0, 1):
                chunk = 2 * o + b
                acc = compute_chunk(b, chunk, acc)
                nxt = chunk + 2

                @pl.when(nxt < nchunk)
                def _():
                    issue(nxt, b)
            return acc

        acc = lax.fori_loop(0, nchunk // 2, outer, ones_f)
        if nchunk % 2:
            acc = compute_chunk(0, nchunk - 1, acc)

        if ntail:
            @pl.when(wid == 1)
            def _():
                pltpu.sync_copy(tail_hbm, first_v)

            tacc = ones_f
            for k in range(128 - ntail, 128, L):
                tacc = tacc * group_factor(first_v, n - 128, k)
            acc = acc * jnp.where(
                jnp.broadcast_to(wid == 1, (L,)), tacc, ones_f)

        for s in (8, 4, 2, 1):
            red_v[...] = acc
            acc = acc * plsc.load_gather(red_v, [lanes ^ s])
        red_v[...] = acc
        pltpu.sync_copy(red_v, parts_hbm.at[pl.ds(wid * L, L)])

        @pl.when(wid == 0)
        def _():
            pltpu.sync_copy(ntp_hbm, ntp_v)
            a = ntp_v[pl.ds(0, L)]
            b = ntp_v[pl.ds(L, L)]
            sa = jnp.sum(a)
            s_tot = sa + jnp.sum(b)
            u = _uniform01(jnp.full((L,), n + 12345, jnp.int32))
            t = u * jnp.broadcast_to(s_tot, (L,))
            ca = plsc.cumsum(a)
            cb = plsc.cumsum(b) + jnp.broadcast_to(sa, (L,))
            cnt = (jnp.sum(jnp.where(ca < t, ones_i, zeros_i))
                   + jnp.sum(jnp.where(cb < t, ones_i, zeros_i)))
            idx = jnp.minimum(jnp.broadcast_to(cnt, (L,)), 2 * L - 1)
            sel = plsc.load_gather(ntp_v, [idx])
            scal_v[...] = sel / (jnp.broadcast_to(s_tot, (L,)) + EPS)
            pltpu.sync_copy(scal_v, pnode_hbm)

    return sc_call


def kernel(node_type_probs, edge_type_probs, w, node):
    n, et = edge_type_probs.shape
    edge_t = edge_type_probs.T
    tail = lax.slice(edge_t, (0, n - 128), (L, n))
    parts, pnode = _make_sc_call(n, et)(edge_t, node_type_probs, tail)
    p_edges = jnp.prod(parts)
    p_O_v = p_edges * pnode[0] + EPS
    w_k = w[node]
    n_i = jnp.float32(n)
    traj_len_minus_1 = jnp.float32(n)
    return n_i / traj_len_minus_1 * jnp.log(p_O_v) * w_k / jnp.float32(4)


# speedup vs baseline: 1.1091x; 1.1091x over previous
import functools

import jax
import jax.numpy as jnp
from jax import lax
from jax.experimental import pallas as pl
from jax.experimental.pallas import tpu as pltpu
from jax.experimental.pallas import tpu_sc as plsc

EPS = 1e-10
L = 16
NC = 2
NS = 16
NW = NC * NS
C = 256


def _uniform01(x):
    x = x ^ lax.shift_right_logical(x, 16)
    x = x * jnp.int32(-2048144789)
    x = x ^ lax.shift_right_logical(x, 13)
    x = x * jnp.int32(-1028477387)
    x = x ^ lax.shift_right_logical(x, 16)
    bits = lax.shift_right_logical(x, 9) | jnp.int32(0x3F800000)
    return plsc.bitcast(bits, jnp.float32) - jnp.float32(1.0)


def _make_sc_call(n, et):
    assert et == L
    rpt = ((n + NW - 1) // NW + C - 1) // C * C
    nchunk = rpt // C
    groups = C // L
    last_base = (n - rpt) // 128 * 128
    tail_base = last_base + rpt
    ntail = n - tail_base
    assert 0 <= ntail < 128 and ntail % (2 * L) == 0 and groups % 4 == 0
    mesh = plsc.VectorSubcoreMesh(
        core_axis_name="c", subcore_axis_name="s", num_cores=NC,
        num_subcores=NS)

    @functools.partial(
        pl.kernel,
        out_type=(
            jax.ShapeDtypeStruct((NW * L,), jnp.float32),
            jax.ShapeDtypeStruct((L,), jnp.float32),
        ),
        mesh=mesh,
        compiler_params=pltpu.CompilerParams(needs_layout_passes=False),
        scratch_types=[
            pltpu.VMEM((L, 2 * C), jnp.float32),
            pltpu.VMEM((L, 128), jnp.float32),
            pltpu.VMEM((2 * L,), jnp.float32),
            pltpu.VMEM((2 * L,), jnp.float32),
            pltpu.VMEM((L,), jnp.float32),
            pltpu.VMEM((L,), jnp.float32),
            pltpu.SemaphoreType.DMA,
            pltpu.SemaphoreType.DMA,
        ],
    )
    def sc_call(edge_hbm, ntp_hbm, tail_hbm,
                parts_hbm, pnode_hbm,
                ring, first_v, table_v, ntp_v, red_v,
                scal_v, sem0, sem1):
        wid = lax.axis_index("s") * NC + lax.axis_index("c")
        base = jnp.minimum(wid * rpt, last_base)
        sems = (sem0, sem1)

        def half_off(chunk):
            return pl.multiple_of((chunk % 2) * C, 128)

        def issue(chunk, sem):
            pltpu.async_copy(
                edge_hbm.at[:, pl.ds(base + chunk * C, C)],
                ring.at[:, pl.ds(half_off(chunk), C)], sem)

        def wait(chunk, sem):
            pltpu.make_async_copy(
                edge_hbm.at[:, pl.ds(0, C)],
                ring.at[:, pl.ds(half_off(chunk), C)], sem).wait()

        issue(0, sem0)
        issue(1, sem1)
        pltpu.sync_copy(edge_hbm.at[:, pl.ds(0, 128)], first_v)

        lanes = lax.iota(jnp.int32, L)
        ones_f = jnp.ones((L,), jnp.float32)
        ones_i = jnp.ones((L,), jnp.int32)
        zeros_i = jnp.zeros((L,), jnp.int32)

        colsum = jnp.zeros((L,), jnp.float32)
        colprod = ones_f
        for j in range(L):
            c = first_v[j, pl.ds(0, L)]
            colsum = colsum + c
            colprod = colprod * c
        rinv = jnp.float32(1.0) / (colsum + EPS)
        r2 = rinv * rinv
        r4 = r2 * r2
        r8 = r4 * r4
        table_v[pl.ds(0, L)] = colprod * (r8 * r8)
        table_v[pl.ds(L, L)] = ones_f

        def group_factor(buf, rowbase, off):
            u = _uniform01(rowbase + off + lanes)
            cols = [buf[j, pl.ds(off, L)] for j in range(L)]
            prefs = [cols[0]]
            for j in range(1, L):
                prefs.append(prefs[-1] + cols[j])
            t = u * prefs[-1]
            bits = [jnp.where(pj < t, ones_i, zeros_i) for pj in prefs]
            while len(bits) > 1:
                bits = [bits[j] + bits[j + 1] for j in range(0, len(bits), 2)]
            return plsc.load_gather(table_v, [bits[0]])

        def pair_body(buf, rowbase, off0, g2, acc):
            f0 = group_factor(buf, rowbase, off0 + (2 * g2) * L)
            f1 = group_factor(buf, rowbase, off0 + (2 * g2 + 1) * L)
            return acc * (f0 * f1)

        def chunk_step(chunk, acc):
            @pl.when(chunk % 2 == 0)
            def _():
                wait(chunk, sem0)

            @pl.when(chunk % 2 == 1)
            def _():
                wait(chunk, sem1)

            rowbase = base + chunk * C
            off0 = half_off(chunk)
            acc = lax.fori_loop(
                0, groups // 2,
                lambda g2, a: pair_body(ring, rowbase, off0, g2, a), acc)
            nxt = chunk + 2

            @pl.when((nxt < nchunk) & (chunk % 2 == 0))
            def _():
                issue(nxt, sem0)

            @pl.when((nxt < nchunk) & (chunk % 2 == 1))
            def _():
                issue(nxt, sem1)

            return acc

        acc = lax.fori_loop(0, nchunk, chunk_step, ones_f)

        if ntail:
            @pl.when(wid == 1)
            def _():
                pltpu.sync_copy(tail_hbm, first_v)

            tacc = ones_f
            for k in range(128 - ntail, 128, L):
                tacc = tacc * group_factor(first_v, n - 128, k)
            acc = acc * jnp.where(
                jnp.broadcast_to(wid == 1, (L,)), tacc, ones_f)

        for s in (8, 4, 2, 1):
            red_v[...] = acc
            acc = acc * plsc.load_gather(red_v, [lanes ^ s])
        red_v[...] = acc
        pltpu.sync_copy(red_v, parts_hbm.at[pl.ds(wid * L, L)])

        @pl.when(wid == 0)
        def _():
            pltpu.sync_copy(ntp_hbm, ntp_v)
            a = ntp_v[pl.ds(0, L)]
            b = ntp_v[pl.ds(L, L)]
            sa = jnp.sum(a)
            s_tot = sa + jnp.sum(b)
            u = _uniform01(jnp.full((L,), n + 12345, jnp.int32))
            t = u * jnp.broadcast_to(s_tot, (L,))
            ca = plsc.cumsum(a)
            cb = plsc.cumsum(b) + jnp.broadcast_to(sa, (L,))
            cnt = (jnp.sum(jnp.where(ca < t, ones_i, zeros_i))
                   + jnp.sum(jnp.where(cb < t, ones_i, zeros_i)))
            idx = jnp.minimum(jnp.broadcast_to(cnt, (L,)), 2 * L - 1)
            sel = plsc.load_gather(ntp_v, [idx])
            scal_v[...] = sel / (jnp.broadcast_to(s_tot, (L,)) + EPS)
            pltpu.sync_copy(scal_v, pnode_hbm)

    return sc_call


def kernel(node_type_probs, edge_type_probs, w, node):
    n, et = edge_type_probs.shape
    edge_t = edge_type_probs.T
    tail = lax.slice(edge_t, (0, n - 128), (L, n))
    parts, pnode = _make_sc_call(n, et)(edge_t, node_type_probs, tail)
    p_edges = jnp.prod(parts)
    p_O_v = p_edges * pnode[0] + EPS
    w_k = w[node]
    n_i = jnp.float32(n)
    traj_len_minus_1 = jnp.float32(n)
    return n_i / traj_len_minus_1 * jnp.log(p_O_v) * w_k / jnp.float32(4)


# speedup vs baseline: 1.1149x; 1.0052x over previous
import functools

import jax
import jax.numpy as jnp
from jax import lax
from jax.experimental import pallas as pl
from jax.experimental.pallas import tpu as pltpu
from jax.experimental.pallas import tpu_sc as plsc

EPS = 1e-10
L = 16
NC = 2
NS = 16
NW = NC * NS
C = 1664


def _uniform01(x):
    x = x ^ lax.shift_right_logical(x, 16)
    x = x * jnp.int32(-2048144789)
    x = x ^ lax.shift_right_logical(x, 13)
    x = x * jnp.int32(-1028477387)
    x = x ^ lax.shift_right_logical(x, 16)
    bits = lax.shift_right_logical(x, 9) | jnp.int32(0x3F800000)
    return plsc.bitcast(bits, jnp.float32) - jnp.float32(1.0)


def _make_sc_call(n, et):
    assert et == L
    rpt = ((n + NW - 1) // NW + C - 1) // C * C
    nchunk = rpt // C
    groups = C // L
    last_base = (n - rpt) // 128 * 128
    tail_base = last_base + rpt
    ntail = n - tail_base
    assert 0 <= ntail < 128 and ntail % (2 * L) == 0 and groups % 4 == 0
    mesh = plsc.VectorSubcoreMesh(
        core_axis_name="c", subcore_axis_name="s", num_cores=NC,
        num_subcores=NS)

    @functools.partial(
        pl.kernel,
        out_type=(
            jax.ShapeDtypeStruct((NW * L,), jnp.float32),
            jax.ShapeDtypeStruct((L,), jnp.float32),
        ),
        mesh=mesh,
        compiler_params=pltpu.CompilerParams(needs_layout_passes=False),
        scratch_types=[
            pltpu.VMEM((L, 2 * C), jnp.float32),
            pltpu.VMEM((L, 128), jnp.float32),
            pltpu.VMEM((2 * L,), jnp.float32),
            pltpu.VMEM((2 * L,), jnp.float32),
            pltpu.VMEM((L,), jnp.float32),
            pltpu.VMEM((L,), jnp.float32),
            pltpu.SemaphoreType.DMA,
            pltpu.SemaphoreType.DMA,
        ],
    )
    def sc_call(edge_hbm, ntp_hbm, tail_hbm,
                parts_hbm, pnode_hbm,
                ring, first_v, table_v, ntp_v, red_v,
                scal_v, sem0, sem1):
        wid = lax.axis_index("s") * NC + lax.axis_index("c")
        base = jnp.minimum(wid * rpt, last_base)
        sems = (sem0, sem1)

        def half_off(chunk):
            return pl.multiple_of((chunk % 2) * C, 128)

        def issue(chunk, sem):
            pltpu.async_copy(
                edge_hbm.at[:, pl.ds(base + chunk * C, C)],
                ring.at[:, pl.ds(half_off(chunk), C)], sem)

        def wait(chunk, sem):
            pltpu.make_async_copy(
                edge_hbm.at[:, pl.ds(0, C)],
                ring.at[:, pl.ds(half_off(chunk), C)], sem).wait()

        issue(0, sem0)
        issue(1, sem1)
        pltpu.sync_copy(edge_hbm.at[:, pl.ds(0, 128)], first_v)

        lanes = lax.iota(jnp.int32, L)
        ones_f = jnp.ones((L,), jnp.float32)
        ones_i = jnp.ones((L,), jnp.int32)
        zeros_i = jnp.zeros((L,), jnp.int32)

        colsum = jnp.zeros((L,), jnp.float32)
        colprod = ones_f
        for j in range(L):
            c = first_v[j, pl.ds(0, L)]
            colsum = colsum + c
            colprod = colprod * c
        rinv = jnp.float32(1.0) / (colsum + EPS)
        r2 = rinv * rinv
        r4 = r2 * r2
        r8 = r4 * r4
        table_v[pl.ds(0, L)] = colprod * (r8 * r8)
        table_v[pl.ds(L, L)] = ones_f

        def group_factor(buf, rowbase, off):
            u = _uniform01(rowbase + off + lanes)
            cols = [buf[j, pl.ds(off, L)] for j in range(L)]
            prefs = [cols[0]]
            for j in range(1, L):
                prefs.append(prefs[-1] + cols[j])
            t = u * prefs[-1]
            bits = [jnp.where(pj < t, ones_i, zeros_i) for pj in prefs]
            while len(bits) > 1:
                bits = [bits[j] + bits[j + 1] for j in range(0, len(bits), 2)]
            return plsc.load_gather(table_v, [bits[0]])

        def pair_body(buf, rowbase, off0, g2, acc):
            f0 = group_factor(buf, rowbase, off0 + (2 * g2) * L)
            f1 = group_factor(buf, rowbase, off0 + (2 * g2 + 1) * L)
            return acc * (f0 * f1)

        def chunk_step(chunk, acc):
            @pl.when(chunk % 2 == 0)
            def _():
                wait(chunk, sem0)

            @pl.when(chunk % 2 == 1)
            def _():
                wait(chunk, sem1)

            rowbase = base + chunk * C
            off0 = half_off(chunk)
            acc = lax.fori_loop(
                0, groups // 2,
                lambda g2, a: pair_body(ring, rowbase, off0, g2, a), acc)
            nxt = chunk + 2

            @pl.when((nxt < nchunk) & (chunk % 2 == 0))
            def _():
                issue(nxt, sem0)

            @pl.when((nxt < nchunk) & (chunk % 2 == 1))
            def _():
                issue(nxt, sem1)

            return acc

        acc = lax.fori_loop(0, nchunk, chunk_step, ones_f)

        if ntail:
            @pl.when(wid == 1)
            def _():
                pltpu.sync_copy(tail_hbm, first_v)

            tacc = ones_f
            for k in range(128 - ntail, 128, L):
                tacc = tacc * group_factor(first_v, n - 128, k)
            acc = acc * jnp.where(
                jnp.broadcast_to(wid == 1, (L,)), tacc, ones_f)

        for s in (8, 4, 2, 1):
            red_v[...] = acc
            acc = acc * plsc.load_gather(red_v, [lanes ^ s])
        red_v[...] = acc
        pltpu.sync_copy(red_v, parts_hbm.at[pl.ds(wid * L, L)])

        @pl.when(wid == 0)
        def _():
            pltpu.sync_copy(ntp_hbm, ntp_v)
            a = ntp_v[pl.ds(0, L)]
            b = ntp_v[pl.ds(L, L)]
            sa = jnp.sum(a)
            s_tot = sa + jnp.sum(b)
            u = _uniform01(jnp.full((L,), n + 12345, jnp.int32))
            t = u * jnp.broadcast_to(s_tot, (L,))
            ca = plsc.cumsum(a)
            cb = plsc.cumsum(b) + jnp.broadcast_to(sa, (L,))
            cnt = (jnp.sum(jnp.where(ca < t, ones_i, zeros_i))
                   + jnp.sum(jnp.where(cb < t, ones_i, zeros_i)))
            idx = jnp.minimum(jnp.broadcast_to(cnt, (L,)), 2 * L - 1)
            sel = plsc.load_gather(ntp_v, [idx])
            scal_v[...] = sel / (jnp.broadcast_to(s_tot, (L,)) + EPS)
            pltpu.sync_copy(scal_v, pnode_hbm)

    return sc_call


def kernel(node_type_probs, edge_type_probs, w, node):
    n, et = edge_type_probs.shape
    edge_t = edge_type_probs.T
    tail = lax.slice(edge_t, (0, n - 128), (L, n))
    parts, pnode = _make_sc_call(n, et)(edge_t, node_type_probs, tail)
    p_edges = jnp.prod(parts)
    p_O_v = p_edges * pnode[0] + EPS
    w_k = w[node]
    n_i = jnp.float32(n)
    traj_len_minus_1 = jnp.float32(n)
    return n_i / traj_len_minus_1 * jnp.log(p_O_v) * w_k / jnp.float32(4)
